# Initial kernel scaffold; baseline (speedup 1.0000x reference)
import jax
import jax.numpy as jnp
from jax.experimental import pallas as pl


def kernel(x, edge_index, weight, att, bias):
    raise NotImplementedError("write your pallas kernel here")


# speedup vs baseline: 68.4661x; 68.4661x over previous
import functools

import jax
import jax.numpy as jnp
import numpy as np
from jax import lax
from jax.experimental import pallas as pl
from jax.experimental.pallas import tpu as pltpu
from jax.experimental.pallas import tpu_sc as plsc

N = 10000
E = 320000
D = 128
H = 8
C = 16
HC = H * C
NEG = 0.2

RB = 2000
NBLK = N // RB
NTILES = 32
EPT = E // NTILES
B = 80
NBATCH = EPT // B
NP = N
ROWS_PER_SUB = NP // 16


def _lrelu(t):
    return jnp.maximum(t, NEG * t)


def _k1_body(x_ref, w_ref, a_ref, h_ref, adas_ref, mx_ref):
    i = pl.program_id(0)
    h = jnp.dot(x_ref[...], w_ref[...], preferred_element_type=jnp.float32)
    h_ref[...] = h
    adas = jnp.dot(h, a_ref[...], preferred_element_type=jnp.float32)
    adas_ref[...] = adas
    m = jnp.max(adas, axis=0)
    m = jnp.broadcast_to(m[None, :], (8, 16))

    @pl.when(i == 0)
    def _():
        mx_ref[...] = jnp.full((8, 16), -1e30, jnp.float32)

    mx_ref[...] = jnp.maximum(mx_ref[...], m)


def _run_k1(x, weight, amat):
    return pl.pallas_call(
        _k1_body,
        grid=(NBLK,),
        in_specs=[
            pl.BlockSpec((RB, D), lambda i: (i, 0)),
            pl.BlockSpec((D, HC), lambda i: (0, 0)),
            pl.BlockSpec((HC, 16), lambda i: (0, 0)),
        ],
        out_specs=[
            pl.BlockSpec((RB, HC), lambda i: (i, 0)),
            pl.BlockSpec((RB, 16), lambda i: (i, 0)),
            pl.BlockSpec((8, 16), lambda i: (0, 0)),
        ],
        out_shape=[
            jax.ShapeDtypeStruct((N, HC), jnp.float32),
            jax.ShapeDtypeStruct((N, 16), jnp.float32),
            jax.ShapeDtypeStruct((8, 16), jnp.float32),
        ],
    )(x, weight, amat)


def _k2_body(adas_ref, mx_ref, t1_ref, ps_ref):
    adas = adas_ref[...]
    ad = adas[:, :H]
    asr = adas[:, H:]
    amax = jnp.max(mx_ref[...], axis=0)[H:]
    u = _lrelu(ad + amax[None, :])
    t1_ref[...] = jnp.concatenate([ad, u], axis=1)
    ps_ref[...] = jnp.exp(_lrelu(ad + asr) - u)


def _run_k2(adas, mx):
    return pl.pallas_call(
        _k2_body,
        grid=(NBLK,),
        in_specs=[
            pl.BlockSpec((RB, 16), lambda i: (i, 0)),
            pl.BlockSpec((8, 16), lambda i: (0, 0)),
        ],
        out_specs=[
            pl.BlockSpec((RB, 16), lambda i: (i, 0)),
            pl.BlockSpec((RB, H), lambda i: (i, 0)),
        ],
        out_shape=[
            jax.ShapeDtypeStruct((N, 16), jnp.float32),
            jax.ShapeDtypeStruct((N, H), jnp.float32),
        ],
    )(adas, mx)


def _vperm(vec, idx):
    dn = lax.GatherDimensionNumbers(offset_dims=(), collapsed_slice_dims=(0,),
                                    start_index_map=(0,))
    return lax.gather(vec, idx[:, None], dn, (1,),
                      mode=lax.GatherScatterMode.PROMISE_IN_BOUNDS)


def _splat(x):
    return jnp.broadcast_to(x, (16,)).astype(jnp.int32)


def _sc_body(src_hbm, dst_hbm, adas_hbm, t1_hbm, h_hbm,
             acc_out, asum_out,
             srcv, dstv, adasv, t1v, hv, pv, fv, zbig, zsmall,
             acc_sh, asum_sh, sem1, sem2, sem3):
    cid = lax.axis_index("c")
    sid = lax.axis_index("s")
    wid = cid * 16 + sid

    iota = lax.iota(jnp.int32, 16)
    col8p8 = jnp.bitwise_and(iota, 7) + 8
    lane_lo = iota < 8
    z16 = jnp.zeros((16,), jnp.float32)

    @pl.loop(0, 125)
    def _(r):
        for c in range(8):
            zbig[r, pl.ds(c * 16, 16)] = z16

    @pl.loop(0, ROWS_PER_SUB)
    def _(r):
        zsmall[r] = z16

    for r in range(5):
        pltpu.sync_copy(zbig, acc_sh.at[pl.ds(sid * ROWS_PER_SUB + r * 125, 125)])
    pltpu.sync_copy(zsmall, asum_sh.at[pl.ds(sid * ROWS_PER_SUB, ROWS_PER_SUB)])
    plsc.subcore_barrier()

    @pl.loop(0, NBATCH)
    def _batch(b):
        base = wid * EPT + b * B
        pltpu.sync_copy(src_hbm.at[pl.ds(base, B)], srcv)
        pltpu.sync_copy(dst_hbm.at[pl.ds(base, B)], dstv)
        g1 = pltpu.async_copy(adas_hbm.at[srcv], adasv, sem1)
        g2 = pltpu.async_copy(t1_hbm.at[dstv], t1v, sem2)
        g3 = pltpu.async_copy(h_hbm.at[srcv], hv, sem3)

        @pl.loop(0, B // 16, unroll=5)
        def _(i):
            sv = srcv[pl.ds(i * 16, 16)]
            dv = dstv[pl.ds(i * 16, 16)]
            fv[pl.ds(i * 16, 16)] = jnp.where(sv == dv, 0.0, 1.0)

        g1.wait()
        g2.wait()

        @pl.loop(0, B, unroll=2)
        def _(e):
            t1row = t1v[e]
            adrow = adasv[e]
            asr = _vperm(adrow, col8p8)
            u = _vperm(t1row, col8p8)
            fbase = fv[pl.ds(pl.multiple_of((e // 16) * 16, 16), 16)]
            f = _vperm(fbase, _splat(jnp.bitwise_and(e, 15)))
            t = t1row + asr
            p = jnp.exp(jnp.maximum(t, NEG * t) - u) * f
            pv[e] = jnp.where(lane_lo, p, 0.0)

        pltpu.sync_copy(pv, asum_sh.at[dstv], add=True)

        g3.wait()

        @pl.loop(0, B, unroll=2)
        def _(e):
            prow = pv[e]
            for hh in range(H):
                s = _vperm(prow, _splat(hh))
                hv[e, pl.ds(hh * 16, 16)] = hv[e, pl.ds(hh * 16, 16)] * s

        pltpu.sync_copy(hv, acc_sh.at[dstv], add=True)

    plsc.subcore_barrier()
    r0 = sid * ROWS_PER_SUB
    pltpu.sync_copy(acc_sh.at[pl.ds(r0, ROWS_PER_SUB)],
                    acc_out.at[cid].at[pl.ds(r0, ROWS_PER_SUB)])
    pltpu.sync_copy(asum_sh.at[pl.ds(r0, ROWS_PER_SUB)],
                    asum_out.at[cid].at[pl.ds(r0, ROWS_PER_SUB)])


def _run_sc(srcE, dstE, adas, t1, h):
    mesh = plsc.VectorSubcoreMesh(core_axis_name="c", subcore_axis_name="s")
    f = pl.kernel(
        _sc_body,
        out_type=[
            jax.ShapeDtypeStruct((2, NP, HC), jnp.float32),
            jax.ShapeDtypeStruct((2, NP, 16), jnp.float32),
        ],
        mesh=mesh,
        compiler_params=pltpu.CompilerParams(use_tc_tiling_on_sc=False),
        scratch_types=[
            pltpu.VMEM((B,), jnp.int32),
            pltpu.VMEM((B,), jnp.int32),
            pltpu.VMEM((B, 16), jnp.float32),
            pltpu.VMEM((B, 16), jnp.float32),
            pltpu.VMEM((B, HC), jnp.float32),
            pltpu.VMEM((B, 16), jnp.float32),
            pltpu.VMEM((B,), jnp.float32),
            pltpu.VMEM((125, 128), jnp.float32),
            pltpu.VMEM((ROWS_PER_SUB, 16), jnp.float32),
            pltpu.VMEM_SHARED((NP, HC), jnp.float32),
            pltpu.VMEM_SHARED((NP, 16), jnp.float32),
            pltpu.SemaphoreType.DMA,
            pltpu.SemaphoreType.DMA,
            pltpu.SemaphoreType.DMA,
        ],
    )
    return f(srcE, dstE, adas, t1, h)


def _k3_body(a0_ref, a1_ref, s0_ref, s1_ref, ps_ref, h_ref, r_ref, b_ref,
             out_ref):
    ps = ps_ref[...]
    den = s0_ref[0][:, :H] + s1_ref[0][:, :H] + ps
    den_e = jnp.dot(den, r_ref[...], preferred_element_type=jnp.float32) + 1e-16
    ps_e = jnp.dot(ps, r_ref[...], preferred_element_type=jnp.float32)
    num = a0_ref[0] + a1_ref[0] + ps_e * h_ref[...]
    out_ref[...] = num / den_e + b_ref[...]


def _k3_call(acc, asum, ps, h, rmat, bias2d):
    return pl.pallas_call(
        _k3_body,
        grid=(NBLK,),
        in_specs=[
            pl.BlockSpec((1, RB, HC), lambda i: (0, i, 0)),
            pl.BlockSpec((1, RB, HC), lambda i: (1, i, 0)),
            pl.BlockSpec((1, RB, 16), lambda i: (0, i, 0)),
            pl.BlockSpec((1, RB, 16), lambda i: (1, i, 0)),
            pl.BlockSpec((RB, H), lambda i: (i, 0)),
            pl.BlockSpec((RB, HC), lambda i: (i, 0)),
            pl.BlockSpec((H, HC), lambda i: (0, 0)),
            pl.BlockSpec((1, HC), lambda i: (0, 0)),
        ],
        out_specs=pl.BlockSpec((RB, HC), lambda i: (i, 0)),
        out_shape=jax.ShapeDtypeStruct((N, HC), jnp.float32),
    )(acc, acc, asum, asum, ps, h, rmat, bias2d)


def kernel(x, edge_index, weight, att, bias):
    att2 = att.reshape(H, 2 * C)
    ad_w = att2[:, :C]
    as_w = att2[:, C:]
    amat = jnp.zeros((HC, 16), jnp.float32)
    rows = jnp.arange(HC)
    amat = amat.at[rows, rows // C].set(ad_w.reshape(-1))
    amat = amat.at[rows, 8 + rows // C].set(as_w.reshape(-1))
    rmat = jnp.kron(jnp.eye(H, dtype=jnp.float32),
                    jnp.ones((1, C), jnp.float32))
    bias2d = bias.reshape(1, HC)
    srcE = edge_index[0]
    dstE = edge_index[1]

    h, adas, mx = _run_k1(x, weight, amat)
    t1, ps = _run_k2(adas, mx)
    acc, asum = _run_sc(srcE, dstE, adas, t1, h)
    out = _k3_call(acc, asum, ps, h, rmat, bias2d)
    return out


# speedup vs baseline: 81.2349x; 1.1865x over previous
import jax
import jax.numpy as jnp
from jax import lax
from jax.experimental import pallas as pl
from jax.experimental.pallas import tpu as pltpu
from jax.experimental.pallas import tpu_sc as plsc

N = 10000
E = 320000
D = 128
H = 8
C = 16
HC = H * C
HH = 64
HS = HH + 16
NEG = 0.2

RB = 2000
NBLK = N // RB
EPT = E // 16
B = 80
NBATCH = EPT // B
ROWS_PER_SUB = N // 16


def _lrelu(t):
    return jnp.maximum(t, NEG * t)


def _k1_body(x_ref, wc_ref, wa_ref, hs_ref, mx_ref):
    c = pl.program_id(0)
    i = pl.program_id(1)
    hh = jnp.dot(x_ref[...], wc_ref[0], preferred_element_type=jnp.float32)
    adas = jnp.dot(x_ref[...], wa_ref[...], preferred_element_type=jnp.float32)
    hs_ref[0] = jnp.concatenate([hh, adas], axis=1)
    m = jnp.broadcast_to(jnp.max(adas, axis=0)[None, :], (8, 16))

    @pl.when(jnp.logical_and(c == 0, i == 0))
    def _():
        mx_ref[...] = jnp.full((8, 16), -1e30, jnp.float32)

    mx_ref[...] = jnp.maximum(mx_ref[...], m)


def _run_k1(x, wsplit, wa):
    return pl.pallas_call(
        _k1_body,
        grid=(2, NBLK),
        in_specs=[
            pl.BlockSpec((RB, D), lambda c, i: (i, 0)),
            pl.BlockSpec((1, D, HH), lambda c, i: (c, 0, 0)),
            pl.BlockSpec((D, 16), lambda c, i: (0, 0)),
        ],
        out_specs=[
            pl.BlockSpec((1, RB, HS), lambda c, i: (c, i, 0)),
            pl.BlockSpec((8, 16), lambda c, i: (0, 0)),
        ],
        out_shape=[
            jax.ShapeDtypeStruct((2, N, HS), jnp.float32),
            jax.ShapeDtypeStruct((8, 16), jnp.float32),
        ],
    )(x, wsplit, wa)


def _k2_body(hs_ref, mx_ref, t1_ref, ps_ref):
    adas = hs_ref[0][:, HH:]
    ad = adas[:, :H]
    asr = adas[:, H:]
    amax = jnp.max(mx_ref[...], axis=0)[H:]
    u = _lrelu(ad + amax[None, :])
    t1_ref[...] = jnp.concatenate([ad, u], axis=1)
    ps_ref[...] = jnp.exp(_lrelu(ad + asr) - u)


def _run_k2(hs, mx):
    return pl.pallas_call(
        _k2_body,
        grid=(NBLK,),
        in_specs=[
            pl.BlockSpec((1, RB, HS), lambda i: (0, i, 0)),
            pl.BlockSpec((8, 16), lambda i: (0, 0)),
        ],
        out_specs=[
            pl.BlockSpec((RB, 16), lambda i: (i, 0)),
            pl.BlockSpec((RB, H), lambda i: (i, 0)),
        ],
        out_shape=[
            jax.ShapeDtypeStruct((N, 16), jnp.float32),
            jax.ShapeDtypeStruct((N, H), jnp.float32),
        ],
    )(hs, mx)


def _vperm(vec, idx):
    dn = lax.GatherDimensionNumbers(offset_dims=(), collapsed_slice_dims=(0,),
                                    start_index_map=(0,))
    return lax.gather(vec, idx[:, None], dn, (1,),
                      mode=lax.GatherScatterMode.PROMISE_IN_BOUNDS)


def _splat(x):
    return jnp.broadcast_to(x, (16,)).astype(jnp.int32)


def _sc_body(src_hbm, dst_hbm, hsA_hbm, hsB_hbm, t1_hbm,
             acc_out, asum_out,
             srcv0, srcv1, dstv0, dstv1, hwv0, hwv1, t1v0, t1v1,
             msgv, pv, fv, zbig, zsmall,
             acc_sh, asum_sh, sem0, sem1):
    cid = lax.axis_index("c")
    sid = lax.axis_index("s")

    iota = lax.iota(jnp.int32, 16)
    col8p8 = jnp.bitwise_and(iota, 7) + 8
    lane_lo = iota < 8
    z16 = jnp.zeros((16,), jnp.float32)

    @pl.loop(0, 125)
    def _(r):
        for c in range(HH // 16):
            zbig[r, pl.ds(c * 16, 16)] = z16

    @pl.loop(0, ROWS_PER_SUB)
    def _(r):
        zsmall[r] = z16

    for r in range(5):
        pltpu.sync_copy(zbig, acc_sh.at[pl.ds(sid * ROWS_PER_SUB + r * 125, 125)])
    pltpu.sync_copy(zsmall, asum_sh.at[pl.ds(sid * ROWS_PER_SUB, ROWS_PER_SUB)])
    plsc.subcore_barrier()

    srcv = (srcv0, srcv1)
    dstv = (dstv0, dstv1)
    hwv = (hwv0, hwv1)
    t1v = (t1v0, t1v1)
    sem = (sem0, sem1)

    def issue(k, g):
        base = sid * EPT + g * B
        pltpu.sync_copy(src_hbm.at[pl.ds(base, B)], srcv[k])
        pltpu.sync_copy(dst_hbm.at[pl.ds(base, B)], dstv[k])

        @pl.when(cid == 0)
        def _():
            pltpu.async_copy(hsA_hbm.at[srcv[k]], hwv[k], sem[k])

        @pl.when(cid == 1)
        def _():
            pltpu.async_copy(hsB_hbm.at[srcv[k]], hwv[k], sem[k])

        pltpu.async_copy(t1_hbm.at[dstv[k]], t1v[k], sem[k])

    def consume(k):
        pltpu.make_async_copy(hsA_hbm.at[srcv[k]], hwv[k], sem[k]).wait()
        pltpu.make_async_copy(t1_hbm.at[dstv[k]], t1v[k], sem[k]).wait()

        for off in list(range(0, B - 16, 16)) + [B - 16]:
            sv = srcv[k][pl.ds(off, 16)]
            dv = dstv[k][pl.ds(off, 16)]
            fv[pl.ds(off, 16)] = jnp.where(sv == dv, 0.0, 1.0)

        @plsc.parallel_loop(0, B, unroll=4)
        def _(e):
            t1row = t1v[k][e]
            adrow = hwv[k][e, pl.ds(HH, 16)]
            asr = _vperm(adrow, col8p8)
            u = _vperm(t1row, col8p8)
            off = pl.multiple_of(jnp.minimum((e // 16) * 16, B - 16), 8)
            fbase = fv[pl.ds(off, 16)]
            f = _vperm(fbase, _splat(e - off))
            t = t1row + asr
            p = jnp.exp(jnp.maximum(t, NEG * t) - u) * f
            pv[e] = jnp.where(lane_lo, p, 0.0)

        @pl.when(cid == 0)
        def _():
            pltpu.sync_copy(pv, asum_sh.at[dstv[k]], add=True)

        @plsc.parallel_loop(0, B, unroll=2)
        def _(e):
            prow = pv[e]
            for hh in range(HH // 16):
                s = _vperm(prow, _splat(4 * cid + hh))
                msgv[e, pl.ds(hh * 16, 16)] = hwv[k][e, pl.ds(hh * 16, 16)] * s

        pltpu.sync_copy(msgv, acc_sh.at[dstv[k]], add=True)

    issue(0, 0)
    issue(1, 1)

    @pl.loop(0, NBATCH, step=2)
    def _pair(g):
        consume(0)

        @pl.when(g + 2 < NBATCH)
        def _():
            issue(0, g + 2)

        consume(1)

        @pl.when(g + 3 < NBATCH)
        def _():
            issue(1, g + 3)

    plsc.subcore_barrier()
    r0 = sid * ROWS_PER_SUB
    pltpu.sync_copy(acc_sh.at[pl.ds(r0, ROWS_PER_SUB)],
                    acc_out.at[cid].at[pl.ds(r0, ROWS_PER_SUB)])

    @pl.when(cid == 0)
    def _():
        pltpu.sync_copy(asum_sh.at[pl.ds(r0, ROWS_PER_SUB)],
                        asum_out.at[pl.ds(r0, ROWS_PER_SUB)])


def _run_sc(srcE, dstE, hsA, hsB, t1):
    mesh = plsc.VectorSubcoreMesh(core_axis_name="c", subcore_axis_name="s")
    f = pl.kernel(
        _sc_body,
        out_type=[
            jax.ShapeDtypeStruct((2, N, HH), jnp.float32),
            jax.ShapeDtypeStruct((N, 16), jnp.float32),
        ],
        mesh=mesh,
        compiler_params=pltpu.CompilerParams(use_tc_tiling_on_sc=False),
        scratch_types=[
            pltpu.VMEM((B,), jnp.int32),
            pltpu.VMEM((B,), jnp.int32),
            pltpu.VMEM((B,), jnp.int32),
            pltpu.VMEM((B,), jnp.int32),
            pltpu.VMEM((B, HS), jnp.float32),
            pltpu.VMEM((B, HS), jnp.float32),
            pltpu.VMEM((B, 16), jnp.float32),
            pltpu.VMEM((B, 16), jnp.float32),
            pltpu.VMEM((B, HH), jnp.float32),
            pltpu.VMEM((B, 16), jnp.float32),
            pltpu.VMEM((B,), jnp.float32),
            pltpu.VMEM((125, HH), jnp.float32),
            pltpu.VMEM((ROWS_PER_SUB, 16), jnp.float32),
            pltpu.VMEM_SHARED((N, HH), jnp.float32),
            pltpu.VMEM_SHARED((N, 16), jnp.float32),
            pltpu.SemaphoreType.DMA,
            pltpu.SemaphoreType.DMA,
        ],
    )
    return f(srcE, dstE, hsA, hsB, t1)


def _k3_body(a0_ref, a1_ref, s_ref, ps_ref, hsA_ref, hsB_ref, r_ref, b_ref,
             out_ref):
    ps = ps_ref[...]
    den = s_ref[...][:, :H] + ps
    den_e = jnp.dot(den, r_ref[...], preferred_element_type=jnp.float32) + 1e-16
    ps_e = jnp.dot(ps, r_ref[...], preferred_element_type=jnp.float32)
    h = jnp.concatenate([hsA_ref[0][:, :HH], hsB_ref[0][:, :HH]], axis=1)
    acc = jnp.concatenate([a0_ref[0], a1_ref[0]], axis=1)
    num = acc + ps_e * h
    out_ref[...] = num / den_e + b_ref[...]


def _k3_call(acc, asum, ps, hs, rmat, bias2d):
    return pl.pallas_call(
        _k3_body,
        grid=(NBLK,),
        in_specs=[
            pl.BlockSpec((1, RB, HH), lambda i: (0, i, 0)),
            pl.BlockSpec((1, RB, HH), lambda i: (1, i, 0)),
            pl.BlockSpec((RB, 16), lambda i: (i, 0)),
            pl.BlockSpec((RB, H), lambda i: (i, 0)),
            pl.BlockSpec((1, RB, HS), lambda i: (0, i, 0)),
            pl.BlockSpec((1, RB, HS), lambda i: (1, i, 0)),
            pl.BlockSpec((H, HC), lambda i: (0, 0)),
            pl.BlockSpec((1, HC), lambda i: (0, 0)),
        ],
        out_specs=pl.BlockSpec((RB, HC), lambda i: (i, 0)),
        out_shape=jax.ShapeDtypeStruct((N, HC), jnp.float32),
    )(acc, acc, asum, ps, hs, hs, rmat, bias2d)


def kernel(x, edge_index, weight, att, bias):
    att2 = att.reshape(H, 2 * C)
    amat = jnp.zeros((HC, 16), jnp.float32)
    rows = jnp.arange(HC)
    amat = amat.at[rows, rows // C].set(att2[:, :C].reshape(-1))
    amat = amat.at[rows, 8 + rows // C].set(att2[:, C:].reshape(-1))
    wa = jnp.dot(weight, amat)
    wsplit = weight.reshape(D, 2, HH).transpose(1, 0, 2)
    rmat = jnp.kron(jnp.eye(H, dtype=jnp.float32),
                    jnp.ones((1, C), jnp.float32))
    bias2d = bias.reshape(1, HC)
    srcE = edge_index[0]
    dstE = edge_index[1]

    hs, mx = _run_k1(x, wsplit, wa)
    t1, ps = _run_k2(hs, mx)
    acc, asum = _run_sc(srcE, dstE, hs[0], hs[1], t1)
    out = _k3_call(acc, asum, ps, hs, rmat, bias2d)
    return out


# speedup vs baseline: 81.4230x; 1.0023x over previous
import jax
import jax.numpy as jnp
from jax import lax
from jax.experimental import pallas as pl
from jax.experimental.pallas import tpu as pltpu
from jax.experimental.pallas import tpu_sc as plsc

N = 10000
E = 320000
D = 128
H = 8
C = 16
HC = H * C
HH = 64
HS = HH + 16
NEG = 0.2

RB = 2000
NBLK = N // RB
EPT = E // 16
B = 80
NBATCH = EPT // B
ROWS_PER_SUB = N // 16


def _lrelu(t):
    return jnp.maximum(t, NEG * t)


def _k1_body(x_ref, wc_ref, wa_ref, hs_ref, mx_ref):
    c = pl.program_id(0)
    i = pl.program_id(1)
    hh = jnp.dot(x_ref[...], wc_ref[0], preferred_element_type=jnp.float32)
    adas = jnp.dot(x_ref[...], wa_ref[...], preferred_element_type=jnp.float32)
    hs_ref[0] = jnp.concatenate([hh, adas], axis=1)
    m = jnp.broadcast_to(jnp.max(adas, axis=0)[None, :], (8, 16))

    @pl.when(jnp.logical_and(c == 0, i == 0))
    def _():
        mx_ref[...] = jnp.full((8, 16), -1e30, jnp.float32)

    mx_ref[...] = jnp.maximum(mx_ref[...], m)


def _run_k1(x, wsplit, wa):
    return pl.pallas_call(
        _k1_body,
        grid=(2, NBLK),
        in_specs=[
            pl.BlockSpec((RB, D), lambda c, i: (i, 0)),
            pl.BlockSpec((1, D, HH), lambda c, i: (c, 0, 0)),
            pl.BlockSpec((D, 16), lambda c, i: (0, 0)),
        ],
        out_specs=[
            pl.BlockSpec((1, RB, HS), lambda c, i: (c, i, 0)),
            pl.BlockSpec((8, 16), lambda c, i: (0, 0)),
        ],
        out_shape=[
            jax.ShapeDtypeStruct((2, N, HS), jnp.float32),
            jax.ShapeDtypeStruct((8, 16), jnp.float32),
        ],
    )(x, wsplit, wa)


def _k2_body(hs_ref, mx_ref, t1_ref, ps_ref):
    adas = hs_ref[0][:, HH:]
    ad = adas[:, :H]
    asr = adas[:, H:]
    amax = jnp.max(mx_ref[...], axis=0)[H:]
    u = _lrelu(ad + amax[None, :])
    t1_ref[...] = jnp.concatenate([ad, u], axis=1)
    ps_ref[...] = jnp.exp(_lrelu(ad + asr) - u)


def _run_k2(hs, mx):
    return pl.pallas_call(
        _k2_body,
        grid=(NBLK,),
        in_specs=[
            pl.BlockSpec((1, RB, HS), lambda i: (0, i, 0)),
            pl.BlockSpec((8, 16), lambda i: (0, 0)),
        ],
        out_specs=[
            pl.BlockSpec((RB, 16), lambda i: (i, 0)),
            pl.BlockSpec((RB, H), lambda i: (i, 0)),
        ],
        out_shape=[
            jax.ShapeDtypeStruct((N, 16), jnp.float32),
            jax.ShapeDtypeStruct((N, H), jnp.float32),
        ],
    )(hs, mx)


def _vperm(vec, idx):
    dn = lax.GatherDimensionNumbers(offset_dims=(), collapsed_slice_dims=(0,),
                                    start_index_map=(0,))
    return lax.gather(vec, idx[:, None], dn, (1,),
                      mode=lax.GatherScatterMode.PROMISE_IN_BOUNDS)


def _splat(x):
    return jnp.broadcast_to(x, (16,)).astype(jnp.int32)


def _sc_body(src_hbm, dst_hbm, hsA_hbm, hsB_hbm, t1_hbm,
             acc_out, asum_out,
             srcv0, srcv1, dstv0, dstv1, hwv0, hwv1, t1v0, t1v1,
             msgv, pv, fv, zbig, zsmall,
             acc_sh, asum_sh, sem0, sem1):
    cid = lax.axis_index("c")
    sid = lax.axis_index("s")

    iota = lax.iota(jnp.int32, 16)
    col8p8 = jnp.bitwise_and(iota, 7) + 8
    z16 = jnp.zeros((16,), jnp.float32)

    @pl.loop(0, 125)
    def _(r):
        for c in range(HH // 16):
            zbig[r, pl.ds(c * 16, 16)] = z16

    @pl.loop(0, ROWS_PER_SUB)
    def _(r):
        zsmall[r] = z16

    for r in range(5):
        pltpu.sync_copy(zbig, acc_sh.at[pl.ds(sid * ROWS_PER_SUB + r * 125, 125)])
    pltpu.sync_copy(zsmall, asum_sh.at[pl.ds(sid * ROWS_PER_SUB, ROWS_PER_SUB)])
    plsc.subcore_barrier()

    srcv = (srcv0, srcv1)
    dstv = (dstv0, dstv1)
    hwv = (hwv0, hwv1)
    t1v = (t1v0, t1v1)
    sem = (sem0, sem1)

    def issue(k, g):
        base = sid * EPT + g * B
        pltpu.sync_copy(src_hbm.at[pl.ds(base, B)], srcv[k])
        pltpu.sync_copy(dst_hbm.at[pl.ds(base, B)], dstv[k])

        @pl.when(cid == 0)
        def _():
            pltpu.async_copy(hsA_hbm.at[srcv[k]], hwv[k], sem[k])

        @pl.when(cid == 1)
        def _():
            pltpu.async_copy(hsB_hbm.at[srcv[k]], hwv[k], sem[k])

        pltpu.async_copy(t1_hbm.at[dstv[k]], t1v[k], sem[k])

    def consume(k):
        pltpu.make_async_copy(hsA_hbm.at[srcv[k]], hwv[k], sem[k]).wait()
        pltpu.make_async_copy(t1_hbm.at[dstv[k]], t1v[k], sem[k]).wait()

        for off in list(range(0, B - 16, 16)) + [B - 16]:
            sv = srcv[k][pl.ds(off, 16)]
            dv = dstv[k][pl.ds(off, 16)]
            fv[pl.ds(off, 16)] = jnp.where(sv == dv, 0.0, 1.0)

        @plsc.parallel_loop(0, B, unroll=4)
        def _(e):
            t1row = t1v[k][e]
            adrow = hwv[k][e, pl.ds(HH, 16)]
            asr = _vperm(adrow, col8p8)
            u = _vperm(t1row, col8p8)
            off = pl.multiple_of(jnp.minimum((e // 16) * 16, B - 16), 8)
            fbase = fv[pl.ds(off, 16)]
            f = _vperm(fbase, _splat(e - off))
            t = t1row + asr
            pv[e] = jnp.exp(jnp.maximum(t, NEG * t) - u) * f

        @pl.when(cid == 0)
        def _():
            pltpu.sync_copy(pv, asum_sh.at[dstv[k]], add=True)

        @plsc.parallel_loop(0, B, unroll=4)
        def _(e):
            prow = pv[e]
            for hh in range(HH // 16):
                s = _vperm(prow, _splat(4 * cid + hh))
                msgv[e, pl.ds(hh * 16, 16)] = hwv[k][e, pl.ds(hh * 16, 16)] * s

        pltpu.sync_copy(msgv, acc_sh.at[dstv[k]], add=True)

    issue(0, 0)
    issue(1, 1)

    @pl.loop(0, NBATCH, step=2)
    def _pair(g):
        consume(0)

        @pl.when(g + 2 < NBATCH)
        def _():
            issue(0, g + 2)

        consume(1)

        @pl.when(g + 3 < NBATCH)
        def _():
            issue(1, g + 3)

    plsc.subcore_barrier()
    r0 = sid * ROWS_PER_SUB
    pltpu.sync_copy(acc_sh.at[pl.ds(r0, ROWS_PER_SUB)],
                    acc_out.at[cid].at[pl.ds(r0, ROWS_PER_SUB)])

    @pl.when(cid == 0)
    def _():
        pltpu.sync_copy(asum_sh.at[pl.ds(r0, ROWS_PER_SUB)],
                        asum_out.at[pl.ds(r0, ROWS_PER_SUB)])


def _run_sc(srcE, dstE, hsA, hsB, t1):
    mesh = plsc.VectorSubcoreMesh(core_axis_name="c", subcore_axis_name="s")
    f = pl.kernel(
        _sc_body,
        out_type=[
            jax.ShapeDtypeStruct((2, N, HH), jnp.float32),
            jax.ShapeDtypeStruct((N, 16), jnp.float32),
        ],
        mesh=mesh,
        compiler_params=pltpu.CompilerParams(use_tc_tiling_on_sc=False),
        scratch_types=[
            pltpu.VMEM((B,), jnp.int32),
            pltpu.VMEM((B,), jnp.int32),
            pltpu.VMEM((B,), jnp.int32),
            pltpu.VMEM((B,), jnp.int32),
            pltpu.VMEM((B, HS), jnp.float32),
            pltpu.VMEM((B, HS), jnp.float32),
            pltpu.VMEM((B, 16), jnp.float32),
            pltpu.VMEM((B, 16), jnp.float32),
            pltpu.VMEM((B, HH), jnp.float32),
            pltpu.VMEM((B, 16), jnp.float32),
            pltpu.VMEM((B,), jnp.float32),
            pltpu.VMEM((125, HH), jnp.float32),
            pltpu.VMEM((ROWS_PER_SUB, 16), jnp.float32),
            pltpu.VMEM_SHARED((N, HH), jnp.float32),
            pltpu.VMEM_SHARED((N, 16), jnp.float32),
            pltpu.SemaphoreType.DMA,
            pltpu.SemaphoreType.DMA,
        ],
    )
    return f(srcE, dstE, hsA, hsB, t1)


def _k3_body(a0_ref, a1_ref, s_ref, ps_ref, hsA_ref, hsB_ref, r_ref, b_ref,
             out_ref):
    ps = ps_ref[...]
    den = s_ref[...][:, :H] + ps
    den_e = jnp.dot(den, r_ref[...], preferred_element_type=jnp.float32) + 1e-16
    ps_e = jnp.dot(ps, r_ref[...], preferred_element_type=jnp.float32)
    h = jnp.concatenate([hsA_ref[0][:, :HH], hsB_ref[0][:, :HH]], axis=1)
    acc = jnp.concatenate([a0_ref[0], a1_ref[0]], axis=1)
    num = acc + ps_e * h
    out_ref[...] = num / den_e + b_ref[...]


def _k3_call(acc, asum, ps, hs, rmat, bias2d):
    return pl.pallas_call(
        _k3_body,
        grid=(NBLK,),
        in_specs=[
            pl.BlockSpec((1, RB, HH), lambda i: (0, i, 0)),
            pl.BlockSpec((1, RB, HH), lambda i: (1, i, 0)),
            pl.BlockSpec((RB, 16), lambda i: (i, 0)),
            pl.BlockSpec((RB, H), lambda i: (i, 0)),
            pl.BlockSpec((1, RB, HS), lambda i: (0, i, 0)),
            pl.BlockSpec((1, RB, HS), lambda i: (1, i, 0)),
            pl.BlockSpec((H, HC), lambda i: (0, 0)),
            pl.BlockSpec((1, HC), lambda i: (0, 0)),
        ],
        out_specs=pl.BlockSpec((RB, HC), lambda i: (i, 0)),
        out_shape=jax.ShapeDtypeStruct((N, HC), jnp.float32),
    )(acc, acc, asum, ps, hs, hs, rmat, bias2d)


def kernel(x, edge_index, weight, att, bias):
    att2 = att.reshape(H, 2 * C)
    amat = jnp.zeros((HC, 16), jnp.float32)
    rows = jnp.arange(HC)
    amat = amat.at[rows, rows // C].set(att2[:, :C].reshape(-1))
    amat = amat.at[rows, 8 + rows // C].set(att2[:, C:].reshape(-1))
    wa = jnp.dot(weight, amat)
    wsplit = weight.reshape(D, 2, HH).transpose(1, 0, 2)
    rmat = jnp.kron(jnp.eye(H, dtype=jnp.float32),
                    jnp.ones((1, C), jnp.float32))
    bias2d = bias.reshape(1, HC)
    srcE = edge_index[0]
    dstE = edge_index[1]

    hs, mx = _run_k1(x, wsplit, wa)
    t1, ps = _run_k2(hs, mx)
    acc, asum = _run_sc(srcE, dstE, hs[0], hs[1], t1)
    out = _k3_call(acc, asum, ps, hs, rmat, bias2d)
    return out


# speedup vs baseline: 134.2682x; 1.6490x over previous
import jax
import jax.numpy as jnp
from jax import lax
from jax.experimental import pallas as pl
from jax.experimental.pallas import tpu as pltpu
from jax.experimental.pallas import tpu_sc as plsc

N = 10000
E = 320000
D = 128
H = 8
C = 16
HC = H * C
HH = 64
HS = HH + 16
NEG = 0.2

RB = 2000
NBLK = N // RB
EPT = E // 16
B = 80
NBATCH = EPT // B
ROWS_PER_SUB = N // 16


def _lrelu(t):
    return jnp.maximum(t, NEG * t)


def _k1_body(x_ref, wc_ref, wa_ref, hs_ref, mx_ref):
    c = pl.program_id(0)
    i = pl.program_id(1)
    hh = jnp.dot(x_ref[...], wc_ref[0], preferred_element_type=jnp.float32)
    adas = jnp.dot(x_ref[...], wa_ref[...], preferred_element_type=jnp.float32)
    hs_ref[0] = jnp.concatenate([hh, adas], axis=1)
    m = jnp.broadcast_to(jnp.max(adas, axis=0)[None, :], (8, 16))

    @pl.when(jnp.logical_and(c == 0, i == 0))
    def _():
        mx_ref[...] = jnp.full((8, 16), -1e30, jnp.float32)

    mx_ref[...] = jnp.maximum(mx_ref[...], m)


def _run_k1(x, wsplit, wa):
    return pl.pallas_call(
        _k1_body,
        grid=(2, NBLK),
        in_specs=[
            pl.BlockSpec((RB, D), lambda c, i: (i, 0)),
            pl.BlockSpec((1, D, HH), lambda c, i: (c, 0, 0)),
            pl.BlockSpec((D, 16), lambda c, i: (0, 0)),
        ],
        out_specs=[
            pl.BlockSpec((1, RB, HS), lambda c, i: (c, i, 0)),
            pl.BlockSpec((8, 16), lambda c, i: (0, 0)),
        ],
        out_shape=[
            jax.ShapeDtypeStruct((2, N, HS), jnp.float32),
            jax.ShapeDtypeStruct((8, 16), jnp.float32),
        ],
    )(x, wsplit, wa)


def _k2_body(hs_ref, mx_ref, t1_ref, ps_ref):
    adas = hs_ref[0][:, HH:]
    ad = adas[:, :H]
    asr = adas[:, H:]
    amax = jnp.max(mx_ref[...], axis=0)[H:]
    u = _lrelu(ad + amax[None, :])
    t1_ref[...] = jnp.concatenate([ad, u], axis=1)
    ps_ref[...] = jnp.exp(_lrelu(ad + asr) - u)


def _run_k2(hs, mx):
    return pl.pallas_call(
        _k2_body,
        grid=(NBLK,),
        in_specs=[
            pl.BlockSpec((1, RB, HS), lambda i: (0, i, 0)),
            pl.BlockSpec((8, 16), lambda i: (0, 0)),
        ],
        out_specs=[
            pl.BlockSpec((RB, 16), lambda i: (i, 0)),
            pl.BlockSpec((RB, H), lambda i: (i, 0)),
        ],
        out_shape=[
            jax.ShapeDtypeStruct((N, 16), jnp.float32),
            jax.ShapeDtypeStruct((N, H), jnp.float32),
        ],
    )(hs, mx)


def _vperm(vec, idx):
    dn = lax.GatherDimensionNumbers(offset_dims=(), collapsed_slice_dims=(0,),
                                    start_index_map=(0,))
    return lax.gather(vec, idx[:, None], dn, (1,),
                      mode=lax.GatherScatterMode.PROMISE_IN_BOUNDS)


def _splat(x):
    return jnp.broadcast_to(x, (16,)).astype(jnp.int32)


def _sc_body(src_hbm, dst_hbm, hsA_hbm, hsB_hbm, t1_hbm,
             acc_out, asum_out,
             srcv0, srcv1, dstv0, dstv1, hwv0, hwv1, t1v0, t1v1,
             msgv0, msgv1, pv0, pv1, dstsc0, dstsc1, fv, zbig, zsmall,
             acc_sh, asum_sh, semG0, semG1, semI0, semI1, semS0, semS1):
    cid = lax.axis_index("c")
    sid = lax.axis_index("s")

    iota = lax.iota(jnp.int32, 16)
    col8p8 = jnp.bitwise_and(iota, 7) + 8
    z16 = jnp.zeros((16,), jnp.float32)

    @pl.loop(0, 125)
    def _(r):
        for c in range(HH // 16):
            zbig[r, pl.ds(c * 16, 16)] = z16

    @pl.loop(0, ROWS_PER_SUB)
    def _(r):
        zsmall[r] = z16

    for r in range(5):
        pltpu.sync_copy(zbig, acc_sh.at[pl.ds(sid * ROWS_PER_SUB + r * 125, 125)])
    pltpu.sync_copy(zsmall, asum_sh.at[pl.ds(sid * ROWS_PER_SUB, ROWS_PER_SUB)])
    plsc.subcore_barrier()

    srcv = (srcv0, srcv1)
    dstv = (dstv0, dstv1)
    hwv = (hwv0, hwv1)
    t1v = (t1v0, t1v1)
    pv = (pv0, pv1)
    msgv = (msgv0, msgv1)
    dstsc = (dstsc0, dstsc1)
    semG = (semG0, semG1)
    semI = (semI0, semI1)
    semS = (semS0, semS1)

    def idx_copy(k, g):
        base = sid * EPT + g * B
        pltpu.async_copy(src_hbm.at[pl.ds(base, B)], srcv[k], semI[k])
        pltpu.async_copy(dst_hbm.at[pl.ds(base, B)], dstv[k], semI[k])

    def wait_idx(k):
        pltpu.make_async_copy(src_hbm.at[pl.ds(0, B)], srcv[k], semI[k]).wait()
        pltpu.make_async_copy(dst_hbm.at[pl.ds(0, B)], dstv[k], semI[k]).wait()

    def fire_gathers(k):
        @pl.when(cid == 0)
        def _():
            pltpu.async_copy(hsA_hbm.at[srcv[k]], hwv[k], semG[k])

        @pl.when(cid == 1)
        def _():
            pltpu.async_copy(hsB_hbm.at[srcv[k]], hwv[k], semG[k])

        pltpu.async_copy(t1_hbm.at[dstv[k]], t1v[k], semG[k])

    def wait_gathers(k):
        pltpu.make_async_copy(hsA_hbm.at[srcv[k]], hwv[k], semG[k]).wait()
        pltpu.make_async_copy(t1_hbm.at[dstv[k]], t1v[k], semG[k]).wait()

    def wait_scatters(k):
        pltpu.make_async_copy(msgv[k], acc_sh.at[dstsc[k]], semS[k]).wait()

        @pl.when(cid == 0)
        def _():
            pltpu.make_async_copy(pv[k], asum_sh.at[dstsc[k]], semS[k]).wait()

    def consume(k, g, nextg):
        wait_gathers(k)

        @pl.when(g >= 2)
        def _():
            wait_scatters(k)

        for off in range(0, B, 16):
            dstsc[k][pl.ds(off, 16)] = dstv[k][pl.ds(off, 16)]

        for off in list(range(0, B - 16, 16)) + [B - 16]:
            sv = srcv[k][pl.ds(off, 16)]
            dv = dstv[k][pl.ds(off, 16)]
            fv[pl.ds(off, 16)] = jnp.where(sv == dv, 0.0, 1.0)

        @pl.when(nextg < NBATCH)
        def _():
            idx_copy(k, nextg)

        @plsc.parallel_loop(0, B, unroll=4)
        def _(e):
            t1row = t1v[k][e]
            adrow = hwv[k][e, pl.ds(HH, 16)]
            asr = _vperm(adrow, col8p8)
            u = _vperm(t1row, col8p8)
            off = pl.multiple_of(jnp.minimum((e // 16) * 16, B - 16), 8)
            fbase = fv[pl.ds(off, 16)]
            f = _vperm(fbase, _splat(e - off))
            t = t1row + asr
            pv[k][e] = jnp.exp(jnp.maximum(t, NEG * t) - u) * f

        @pl.when(cid == 0)
        def _():
            pltpu.async_copy(pv[k], asum_sh.at[dstsc[k]], semS[k], add=True)

        @plsc.parallel_loop(0, B, unroll=4)
        def _(e):
            prow = pv[k][e]
            for hh in range(HH // 16):
                s = _vperm(prow, _splat(4 * cid + hh))
                msgv[k][e, pl.ds(hh * 16, 16)] = (
                    hwv[k][e, pl.ds(hh * 16, 16)] * s)

        pltpu.async_copy(msgv[k], acc_sh.at[dstsc[k]], semS[k], add=True)

        @pl.when(nextg < NBATCH)
        def _():
            wait_idx(k)
            fire_gathers(k)

    idx_copy(0, 0)
    idx_copy(1, 1)
    wait_idx(0)
    fire_gathers(0)
    wait_idx(1)
    fire_gathers(1)

    @pl.loop(0, NBATCH, step=2)
    def _pair(g):
        consume(0, g, g + 2)
        consume(1, g + 1, g + 3)

    wait_scatters(0)
    wait_scatters(1)

    plsc.subcore_barrier()
    r0 = sid * ROWS_PER_SUB
    pltpu.sync_copy(acc_sh.at[pl.ds(r0, ROWS_PER_SUB)],
                    acc_out.at[cid].at[pl.ds(r0, ROWS_PER_SUB)])

    @pl.when(cid == 0)
    def _():
        pltpu.sync_copy(asum_sh.at[pl.ds(r0, ROWS_PER_SUB)],
                        asum_out.at[pl.ds(r0, ROWS_PER_SUB)])


def _run_sc(srcE, dstE, hsA, hsB, t1):
    mesh = plsc.VectorSubcoreMesh(core_axis_name="c", subcore_axis_name="s")
    f = pl.kernel(
        _sc_body,
        out_type=[
            jax.ShapeDtypeStruct((2, N, HH), jnp.float32),
            jax.ShapeDtypeStruct((N, 16), jnp.float32),
        ],
        mesh=mesh,
        compiler_params=pltpu.CompilerParams(use_tc_tiling_on_sc=False),
        scratch_types=[
            pltpu.VMEM((B,), jnp.int32),
            pltpu.VMEM((B,), jnp.int32),
            pltpu.VMEM((B,), jnp.int32),
            pltpu.VMEM((B,), jnp.int32),
            pltpu.VMEM((B, HS), jnp.float32),
            pltpu.VMEM((B, HS), jnp.float32),
            pltpu.VMEM((B, 16), jnp.float32),
            pltpu.VMEM((B, 16), jnp.float32),
            pltpu.VMEM((B, HH), jnp.float32),
            pltpu.VMEM((B, HH), jnp.float32),
            pltpu.VMEM((B, 16), jnp.float32),
            pltpu.VMEM((B, 16), jnp.float32),
            pltpu.VMEM((B,), jnp.int32),
            pltpu.VMEM((B,), jnp.int32),
            pltpu.VMEM((B,), jnp.float32),
            pltpu.VMEM((125, HH), jnp.float32),
            pltpu.VMEM((ROWS_PER_SUB, 16), jnp.float32),
            pltpu.VMEM_SHARED((N, HH), jnp.float32),
            pltpu.VMEM_SHARED((N, 16), jnp.float32),
            pltpu.SemaphoreType.DMA,
            pltpu.SemaphoreType.DMA,
            pltpu.SemaphoreType.DMA,
            pltpu.SemaphoreType.DMA,
            pltpu.SemaphoreType.DMA,
            pltpu.SemaphoreType.DMA,
        ],
    )
    return f(srcE, dstE, hsA, hsB, t1)


def _k3_body(a0_ref, a1_ref, s_ref, ps_ref, hsA_ref, hsB_ref, r_ref, b_ref,
             out_ref):
    ps = ps_ref[...]
    den = s_ref[...][:, :H] + ps
    den_e = jnp.dot(den, r_ref[...], preferred_element_type=jnp.float32) + 1e-16
    ps_e = jnp.dot(ps, r_ref[...], preferred_element_type=jnp.float32)
    h = jnp.concatenate([hsA_ref[0][:, :HH], hsB_ref[0][:, :HH]], axis=1)
    acc = jnp.concatenate([a0_ref[0], a1_ref[0]], axis=1)
    num = acc + ps_e * h
    out_ref[...] = num / den_e + b_ref[...]


def _k3_call(acc, asum, ps, hs, rmat, bias2d):
    return pl.pallas_call(
        _k3_body,
        grid=(NBLK,),
        in_specs=[
            pl.BlockSpec((1, RB, HH), lambda i: (0, i, 0)),
            pl.BlockSpec((1, RB, HH), lambda i: (1, i, 0)),
            pl.BlockSpec((RB, 16), lambda i: (i, 0)),
            pl.BlockSpec((RB, H), lambda i: (i, 0)),
            pl.BlockSpec((1, RB, HS), lambda i: (0, i, 0)),
            pl.BlockSpec((1, RB, HS), lambda i: (1, i, 0)),
            pl.BlockSpec((H, HC), lambda i: (0, 0)),
            pl.BlockSpec((1, HC), lambda i: (0, 0)),
        ],
        out_specs=pl.BlockSpec((RB, HC), lambda i: (i, 0)),
        out_shape=jax.ShapeDtypeStruct((N, HC), jnp.float32),
    )(acc, acc, asum, ps, hs, hs, rmat, bias2d)


def kernel(x, edge_index, weight, att, bias):
    att2 = att.reshape(H, 2 * C)
    amat = jnp.zeros((HC, 16), jnp.float32)
    rows = jnp.arange(HC)
    amat = amat.at[rows, rows // C].set(att2[:, :C].reshape(-1))
    amat = amat.at[rows, 8 + rows // C].set(att2[:, C:].reshape(-1))
    wa = jnp.dot(weight, amat)
    wsplit = weight.reshape(D, 2, HH).transpose(1, 0, 2)
    rmat = jnp.kron(jnp.eye(H, dtype=jnp.float32),
                    jnp.ones((1, C), jnp.float32))
    bias2d = bias.reshape(1, HC)
    srcE = edge_index[0]
    dstE = edge_index[1]

    hs, mx = _run_k1(x, wsplit, wa)
    t1, ps = _run_k2(hs, mx)
    acc, asum = _run_sc(srcE, dstE, hs[0], hs[1], t1)
    out = _k3_call(acc, asum, ps, hs, rmat, bias2d)
    return out


# speedup vs baseline: 135.9657x; 1.0126x over previous
import jax
import jax.numpy as jnp
from jax import lax
from jax.experimental import pallas as pl
from jax.experimental.pallas import tpu as pltpu
from jax.experimental.pallas import tpu_sc as plsc

N = 10000
E = 320000
D = 128
H = 8
C = 16
HC = H * C
HH = 64
HS = HH + 16
NEG = 0.2

RB = 2000
NBLK = N // RB
EPT = E // 16
B = 80
NBATCH = EPT // B
ROWS_PER_SUB = N // 16


def _lrelu(t):
    return jnp.maximum(t, NEG * t)


def _k1_body(x_ref, wc_ref, wa_ref, hs_ref, mx_ref):
    c = pl.program_id(0)
    i = pl.program_id(1)
    hh = jnp.dot(x_ref[...], wc_ref[0], preferred_element_type=jnp.float32)
    adas = jnp.dot(x_ref[...], wa_ref[...], preferred_element_type=jnp.float32)
    hs_ref[0] = jnp.concatenate([hh, adas], axis=1)
    m = jnp.broadcast_to(jnp.max(adas, axis=0)[None, :], (8, 16))

    @pl.when(jnp.logical_and(c == 0, i == 0))
    def _():
        mx_ref[...] = jnp.full((8, 16), -1e30, jnp.float32)

    mx_ref[...] = jnp.maximum(mx_ref[...], m)


def _run_k1(x, wsplit, wa):
    return pl.pallas_call(
        _k1_body,
        grid=(2, NBLK),
        in_specs=[
            pl.BlockSpec((RB, D), lambda c, i: (i, 0)),
            pl.BlockSpec((1, D, HH), lambda c, i: (c, 0, 0)),
            pl.BlockSpec((D, 16), lambda c, i: (0, 0)),
        ],
        out_specs=[
            pl.BlockSpec((1, RB, HS), lambda c, i: (c, i, 0)),
            pl.BlockSpec((8, 16), lambda c, i: (0, 0)),
        ],
        out_shape=[
            jax.ShapeDtypeStruct((2, N, HS), jnp.float32),
            jax.ShapeDtypeStruct((8, 16), jnp.float32),
        ],
    )(x, wsplit, wa)


def _k2_body(hs_ref, mx_ref, t1_ref, ps_ref):
    adas = hs_ref[0][:, HH:]
    ad = adas[:, :H]
    asr = adas[:, H:]
    amax = jnp.max(mx_ref[...], axis=0)[H:]
    u = _lrelu(ad + amax[None, :])
    t1_ref[...] = jnp.concatenate([ad, u], axis=1)
    ps_ref[...] = jnp.exp(_lrelu(ad + asr) - u)


def _run_k2(hs, mx):
    return pl.pallas_call(
        _k2_body,
        grid=(NBLK,),
        in_specs=[
            pl.BlockSpec((1, RB, HS), lambda i: (0, i, 0)),
            pl.BlockSpec((8, 16), lambda i: (0, 0)),
        ],
        out_specs=[
            pl.BlockSpec((RB, 16), lambda i: (i, 0)),
            pl.BlockSpec((RB, H), lambda i: (i, 0)),
        ],
        out_shape=[
            jax.ShapeDtypeStruct((N, 16), jnp.float32),
            jax.ShapeDtypeStruct((N, H), jnp.float32),
        ],
    )(hs, mx)


def _vperm(vec, idx):
    dn = lax.GatherDimensionNumbers(offset_dims=(), collapsed_slice_dims=(0,),
                                    start_index_map=(0,))
    return lax.gather(vec, idx[:, None], dn, (1,),
                      mode=lax.GatherScatterMode.PROMISE_IN_BOUNDS)


def _splat(x):
    return jnp.broadcast_to(x, (16,)).astype(jnp.int32)


def _sc_body(src_hbm, dst_hbm, hsA_hbm, hsB_hbm, t1_hbm,
             acc_out, asum_out,
             srcv0, srcv1, dstv0, dstv1, hwv0, hwv1, t1v0, t1v1,
             msgv0, msgv1, pv0, pv1, dstsc0, dstsc1, fv, zbig, zsmall,
             acc_sh, asum_sh, semG0, semG1, semI0, semI1, semS0, semS1):
    cid = lax.axis_index("c")
    sid = lax.axis_index("s")

    iota = lax.iota(jnp.int32, 16)
    col8p8 = jnp.bitwise_and(iota, 7) + 8
    z16 = jnp.zeros((16,), jnp.float32)

    @pl.loop(0, 125)
    def _(r):
        for c in range(HH // 16):
            zbig[r, pl.ds(c * 16, 16)] = z16

    @pl.loop(0, ROWS_PER_SUB)
    def _(r):
        zsmall[r] = z16

    for r in range(5):
        pltpu.sync_copy(zbig, acc_sh.at[pl.ds(sid * ROWS_PER_SUB + r * 125, 125)])
    pltpu.sync_copy(zsmall, asum_sh.at[pl.ds(sid * ROWS_PER_SUB, ROWS_PER_SUB)])
    plsc.subcore_barrier()

    srcv = (srcv0, srcv1)
    dstv = (dstv0, dstv1)
    hwv = (hwv0, hwv1)
    t1v = (t1v0, t1v1)
    pv = (pv0, pv1)
    msgv = (msgv0, msgv1)
    dstsc = (dstsc0, dstsc1)
    semG = (semG0, semG1)
    semI = (semI0, semI1)
    semS = (semS0, semS1)

    def idx_copy(k, g):
        base = sid * EPT + g * B
        pltpu.async_copy(src_hbm.at[pl.ds(base, B)], srcv[k], semI[k])
        pltpu.async_copy(dst_hbm.at[pl.ds(base, B)], dstv[k], semI[k])

    def wait_idx(k):
        pltpu.make_async_copy(src_hbm.at[pl.ds(0, B)], srcv[k], semI[k]).wait()
        pltpu.make_async_copy(dst_hbm.at[pl.ds(0, B)], dstv[k], semI[k]).wait()

    def fire_gathers(k):
        @pl.when(cid == 0)
        def _():
            pltpu.async_copy(hsA_hbm.at[srcv[k]], hwv[k], semG[k])

        @pl.when(cid == 1)
        def _():
            pltpu.async_copy(hsB_hbm.at[srcv[k]], hwv[k], semG[k])

        pltpu.async_copy(t1_hbm.at[dstv[k]], t1v[k], semG[k])

    def wait_gathers(k):
        pltpu.make_async_copy(hsA_hbm.at[srcv[k]], hwv[k], semG[k]).wait()
        pltpu.make_async_copy(t1_hbm.at[dstv[k]], t1v[k], semG[k]).wait()

    def wait_scatters(k):
        pltpu.make_async_copy(msgv[k], acc_sh.at[dstsc[k]], semS[k]).wait()

        @pl.when(cid == 0)
        def _():
            pltpu.make_async_copy(pv[k], asum_sh.at[dstsc[k]], semS[k]).wait()

    def consume(k, g, nextg):
        wait_gathers(k)

        @pl.when(g >= 2)
        def _():
            wait_scatters(k)

        for off in range(0, B, 16):
            dstsc[k][pl.ds(off, 16)] = dstv[k][pl.ds(off, 16)]

        for off in list(range(0, B - 16, 16)) + [B - 16]:
            sv = srcv[k][pl.ds(off, 16)]
            dv = dstv[k][pl.ds(off, 16)]
            fv[pl.ds(off, 16)] = jnp.where(sv == dv, 0.0, 1.0)

        @pl.when(nextg < NBATCH)
        def _():
            idx_copy(k, nextg)

        @plsc.parallel_loop(0, B, unroll=8)
        def _(e):
            t1row = t1v[k][e]
            adrow = hwv[k][e, pl.ds(HH, 16)]
            asr = _vperm(adrow, col8p8)
            u = _vperm(t1row, col8p8)
            off = pl.multiple_of(jnp.minimum((e // 16) * 16, B - 16), 8)
            fbase = fv[pl.ds(off, 16)]
            f = _vperm(fbase, _splat(e - off))
            t = t1row + asr
            p = jnp.exp(jnp.maximum(t, NEG * t) - u) * f
            pv[k][e] = p
            for hh in range(HH // 16):
                s = _vperm(p, _splat(4 * cid + hh))
                msgv[k][e, pl.ds(hh * 16, 16)] = (
                    hwv[k][e, pl.ds(hh * 16, 16)] * s)

        @pl.when(cid == 0)
        def _():
            pltpu.async_copy(pv[k], asum_sh.at[dstsc[k]], semS[k], add=True)

        pltpu.async_copy(msgv[k], acc_sh.at[dstsc[k]], semS[k], add=True)

        @pl.when(nextg < NBATCH)
        def _():
            wait_idx(k)
            fire_gathers(k)

    idx_copy(0, 0)
    idx_copy(1, 1)
    wait_idx(0)
    fire_gathers(0)
    wait_idx(1)
    fire_gathers(1)

    @pl.loop(0, NBATCH, step=2)
    def _pair(g):
        consume(0, g, g + 2)
        consume(1, g + 1, g + 3)

    wait_scatters(0)
    wait_scatters(1)

    plsc.subcore_barrier()
    r0 = sid * ROWS_PER_SUB
    pltpu.sync_copy(acc_sh.at[pl.ds(r0, ROWS_PER_SUB)],
                    acc_out.at[cid].at[pl.ds(r0, ROWS_PER_SUB)])

    @pl.when(cid == 0)
    def _():
        pltpu.sync_copy(asum_sh.at[pl.ds(r0, ROWS_PER_SUB)],
                        asum_out.at[pl.ds(r0, ROWS_PER_SUB)])


def _run_sc(srcE, dstE, hsA, hsB, t1):
    mesh = plsc.VectorSubcoreMesh(core_axis_name="c", subcore_axis_name="s")
    f = pl.kernel(
        _sc_body,
        out_type=[
            jax.ShapeDtypeStruct((2, N, HH), jnp.float32),
            jax.ShapeDtypeStruct((N, 16), jnp.float32),
        ],
        mesh=mesh,
        compiler_params=pltpu.CompilerParams(use_tc_tiling_on_sc=False),
        scratch_types=[
            pltpu.VMEM((B,), jnp.int32),
            pltpu.VMEM((B,), jnp.int32),
            pltpu.VMEM((B,), jnp.int32),
            pltpu.VMEM((B,), jnp.int32),
            pltpu.VMEM((B, HS), jnp.float32),
            pltpu.VMEM((B, HS), jnp.float32),
            pltpu.VMEM((B, 16), jnp.float32),
            pltpu.VMEM((B, 16), jnp.float32),
            pltpu.VMEM((B, HH), jnp.float32),
            pltpu.VMEM((B, HH), jnp.float32),
            pltpu.VMEM((B, 16), jnp.float32),
            pltpu.VMEM((B, 16), jnp.float32),
            pltpu.VMEM((B,), jnp.int32),
            pltpu.VMEM((B,), jnp.int32),
            pltpu.VMEM((B,), jnp.float32),
            pltpu.VMEM((125, HH), jnp.float32),
            pltpu.VMEM((ROWS_PER_SUB, 16), jnp.float32),
            pltpu.VMEM_SHARED((N, HH), jnp.float32),
            pltpu.VMEM_SHARED((N, 16), jnp.float32),
            pltpu.SemaphoreType.DMA,
            pltpu.SemaphoreType.DMA,
            pltpu.SemaphoreType.DMA,
            pltpu.SemaphoreType.DMA,
            pltpu.SemaphoreType.DMA,
            pltpu.SemaphoreType.DMA,
        ],
    )
    return f(srcE, dstE, hsA, hsB, t1)


def _k3_body(a0_ref, a1_ref, s_ref, ps_ref, hsA_ref, hsB_ref, r_ref, b_ref,
             out_ref):
    ps = ps_ref[...]
    den = s_ref[...][:, :H] + ps
    den_e = jnp.dot(den, r_ref[...], preferred_element_type=jnp.float32) + 1e-16
    ps_e = jnp.dot(ps, r_ref[...], preferred_element_type=jnp.float32)
    h = jnp.concatenate([hsA_ref[0][:, :HH], hsB_ref[0][:, :HH]], axis=1)
    acc = jnp.concatenate([a0_ref[0], a1_ref[0]], axis=1)
    num = acc + ps_e * h
    out_ref[...] = num / den_e + b_ref[...]


def _k3_call(acc, asum, ps, hs, rmat, bias2d):
    return pl.pallas_call(
        _k3_body,
        grid=(NBLK,),
        in_specs=[
            pl.BlockSpec((1, RB, HH), lambda i: (0, i, 0)),
            pl.BlockSpec((1, RB, HH), lambda i: (1, i, 0)),
            pl.BlockSpec((RB, 16), lambda i: (i, 0)),
            pl.BlockSpec((RB, H), lambda i: (i, 0)),
            pl.BlockSpec((1, RB, HS), lambda i: (0, i, 0)),
            pl.BlockSpec((1, RB, HS), lambda i: (1, i, 0)),
            pl.BlockSpec((H, HC), lambda i: (0, 0)),
            pl.BlockSpec((1, HC), lambda i: (0, 0)),
        ],
        out_specs=pl.BlockSpec((RB, HC), lambda i: (i, 0)),
        out_shape=jax.ShapeDtypeStruct((N, HC), jnp.float32),
    )(acc, acc, asum, ps, hs, hs, rmat, bias2d)


def kernel(x, edge_index, weight, att, bias):
    att2 = att.reshape(H, 2 * C)
    amat = jnp.zeros((HC, 16), jnp.float32)
    rows = jnp.arange(HC)
    amat = amat.at[rows, rows // C].set(att2[:, :C].reshape(-1))
    amat = amat.at[rows, 8 + rows // C].set(att2[:, C:].reshape(-1))
    wa = jnp.dot(weight, amat)
    wsplit = weight.reshape(D, 2, HH).transpose(1, 0, 2)
    rmat = jnp.kron(jnp.eye(H, dtype=jnp.float32),
                    jnp.ones((1, C), jnp.float32))
    bias2d = bias.reshape(1, HC)
    srcE = edge_index[0]
    dstE = edge_index[1]

    hs, mx = _run_k1(x, wsplit, wa)
    t1, ps = _run_k2(hs, mx)
    acc, asum = _run_sc(srcE, dstE, hs[0], hs[1], t1)
    out = _k3_call(acc, asum, ps, hs, rmat, bias2d)
    return out
